# baseline (speedup 1.0000x reference)
import functools

import jax
import jax.numpy as jnp
from jax import lax
from jax.experimental import pallas as pl
from jax.experimental.pallas import tpu as pltpu
from jax.experimental.pallas import tpu_sc as plsc

N = 10000
E = 160000
D = 256
H = 128
NC = 2
NS = 16
L = 16

K = 128
CH = 80
EPT = CH * K
EPAD = NS * EPT
EPC = EPAD // (NC * NS)

NROW = 624
NPAD = 10112
NZ = NPAD // NS
NPC = 10240

R = 1024


def _sc_mesh():
    return plsc.VectorSubcoreMesh(core_axis_name="c", subcore_axis_name="s")


def _sc_counts(dst_p):
    @functools.partial(
        pl.kernel,
        out_type=jax.ShapeDtypeStruct((NC * NS * NPC,), jnp.float32),
        mesh=_sc_mesh(),
        scratch_types=[
            pltpu.VMEM((EPC,), jnp.int32),
            pltpu.VMEM((NPC,), jnp.float32),
        ],
        compiler_params=pltpu.CompilerParams(needs_layout_passes=False),
    )
    def k(dst_hbm, out_hbm, dstv, cntv):
        c = lax.axis_index("c")
        s = lax.axis_index("s")
        g = s * NC + c
        zeros = jnp.zeros((L,), jnp.float32)

        def zbody(i, carry):
            cntv[pl.ds(i * L, L)] = zeros
            return carry

        lax.fori_loop(0, NPC // L, zbody, 0)
        pltpu.sync_copy(dst_hbm.at[pl.ds(g * EPC, EPC)], dstv)
        ones = jnp.ones((L,), jnp.float32)

        def body(i, carry):
            idx = dstv[pl.ds(i * L, L)]
            plsc.addupdate_scatter(cntv, [idx], ones)
            return carry

        lax.fori_loop(0, EPC // L, body, 0)
        pltpu.sync_copy(cntv, out_hbm.at[pl.ds(g * NPC, NPC)])

    return k(dst_p)


def _sc_segsum(y2n, src_s, dst_s, zrows):
    @functools.partial(
        pl.kernel,
        out_type=jax.ShapeDtypeStruct((N, D), jnp.float32),
        mesh=_sc_mesh(),
        scratch_types=[
            pltpu.VMEM((EPT,), jnp.int32),
            pltpu.VMEM((CH, K), jnp.int32),
            pltpu.VMEM((K, H), jnp.float32),
            pltpu.VMEM_SHARED((NPAD, H), jnp.float32),
        ],
    )
    def k(y_hbm, src_hbm, dst_hbm, z_hbm, out_hbm, sall, dall, rows, acc):
        c = lax.axis_index("c")
        s = lax.axis_index("s")
        base = c * N

        pltpu.sync_copy(src_hbm.at[pl.ds(s * EPT, EPT)], sall)
        pltpu.sync_copy(dst_hbm.at[s], dall)
        pltpu.sync_copy(z_hbm.at[pl.ds(s * NZ, NZ)], acc.at[pl.ds(s * NZ, NZ)])

        def ab(i, carry):
            sl = pl.ds(i * L, L)
            sall[sl] = sall[sl] + base
            return carry

        lax.fori_loop(0, EPT // L, ab, 0)
        plsc.subcore_barrier()

        def step(j, carry):
            pltpu.sync_copy(y_hbm.at[sall.at[pl.ds(j * K, K)]], rows)
            pltpu.sync_copy(rows, acc.at[dall.at[j]], add=True)
            return carry

        lax.fori_loop(0, CH, step, 0)
        plsc.subcore_barrier()
        pltpu.sync_copy(
            acc.at[pl.ds(s * NROW, NROW)],
            out_hbm.at[pl.ds(s * NROW, NROW), pl.ds(c * H, H)],
        )

        @pl.when(s == 0)
        def _tail():
            pltpu.sync_copy(
                acc.at[pl.ds(NS * NROW, N - NS * NROW)],
                out_hbm.at[pl.ds(NS * NROW, N - NS * NROW), pl.ds(c * H, H)],
            )

    return k(y2n, src_s, dst_s, zrows)


def _h_block(a_ref, c_ref, zp_ref):
    cnt = jnp.sum(c_ref[...], axis=0)
    inv = 1.0 / jnp.maximum(cnt, 1.0)
    return jnp.maximum(a_ref[...] * inv[:, None] + zp_ref[...], 0.0)


def _tc_y(agg, cnt32, z_prev, Wl):
    grid = (pl.cdiv(N, R),)

    def body(a_ref, c_ref, zp_ref, wl_ref, y_ref):
        h = _h_block(a_ref, c_ref, zp_ref)
        y = jnp.dot(h, wl_ref[...], preferred_element_type=jnp.float32)
        y_ref[0] = y[:, :H]
        y_ref[1] = y[:, H:]

    return pl.pallas_call(
        body,
        grid=grid,
        in_specs=[
            pl.BlockSpec((R, D), lambda i: (i, 0)),
            pl.BlockSpec((NC * NS, R), lambda i: (0, i)),
            pl.BlockSpec((R, D), lambda i: (i, 0)),
            pl.BlockSpec((D, D), lambda i: (0, 0)),
        ],
        out_specs=pl.BlockSpec((2, R, H), lambda i: (0, i, 0)),
        out_shape=jax.ShapeDtypeStruct((2, N, H), jnp.float32),
    )(agg, cnt32, z_prev, Wl)


def _tc_z(agg, cnt32, z_prev, Wr, bl8):
    grid = (pl.cdiv(N, R),)

    def body(a_ref, c_ref, zp_ref, wr_ref, b_ref, z_ref):
        h = _h_block(a_ref, c_ref, zp_ref)
        z_ref[...] = (
            jnp.dot(h, wr_ref[...], preferred_element_type=jnp.float32)
            + b_ref[0][None, :]
        )

    return pl.pallas_call(
        body,
        grid=grid,
        in_specs=[
            pl.BlockSpec((R, D), lambda i: (i, 0)),
            pl.BlockSpec((NC * NS, R), lambda i: (0, i)),
            pl.BlockSpec((R, D), lambda i: (i, 0)),
            pl.BlockSpec((D, D), lambda i: (0, 0)),
            pl.BlockSpec((8, D), lambda i: (0, 0)),
        ],
        out_specs=pl.BlockSpec((R, D), lambda i: (i, 0)),
        out_shape=jax.ShapeDtypeStruct((N, D), jnp.float32),
    )(agg, cnt32, z_prev, Wr, bl8)


def _tc_xw_halves(x, W):
    grid = (pl.cdiv(N, R),)

    def body(x_ref, w_ref, y_ref):
        y = jnp.dot(x_ref[...], w_ref[...], preferred_element_type=jnp.float32)
        y_ref[0] = y[:, :H]
        y_ref[1] = y[:, H:]

    return pl.pallas_call(
        body,
        grid=grid,
        in_specs=[
            pl.BlockSpec((R, D), lambda i: (i, 0)),
            pl.BlockSpec((D, D), lambda i: (0, 0)),
        ],
        out_specs=pl.BlockSpec((2, R, H), lambda i: (0, i, 0)),
        out_shape=jax.ShapeDtypeStruct((2, N, H), jnp.float32),
    )(x, W)


def _tc_xw_bias(x, W, bl8):
    grid = (pl.cdiv(N, R),)

    def body(x_ref, w_ref, b_ref, z_ref):
        z_ref[...] = (
            jnp.dot(x_ref[...], w_ref[...], preferred_element_type=jnp.float32)
            + b_ref[0][None, :]
        )

    return pl.pallas_call(
        body,
        grid=grid,
        in_specs=[
            pl.BlockSpec((R, D), lambda i: (i, 0)),
            pl.BlockSpec((D, D), lambda i: (0, 0)),
            pl.BlockSpec((8, D), lambda i: (0, 0)),
        ],
        out_specs=pl.BlockSpec((R, D), lambda i: (i, 0)),
        out_shape=jax.ShapeDtypeStruct((N, D), jnp.float32),
    )(x, W, bl8)


def _tc_last(agg, cnt32, z_prev):
    grid = (pl.cdiv(N, R),)

    def body(a_ref, c_ref, zp_ref, o_ref):
        cnt = jnp.sum(c_ref[...], axis=0)
        inv = 1.0 / jnp.maximum(cnt, 1.0)
        o_ref[...] = a_ref[...] * inv[:, None] + zp_ref[...]

    return pl.pallas_call(
        body,
        grid=grid,
        in_specs=[
            pl.BlockSpec((R, D), lambda i: (i, 0)),
            pl.BlockSpec((NC * NS, R), lambda i: (0, i)),
            pl.BlockSpec((R, D), lambda i: (i, 0)),
        ],
        out_specs=pl.BlockSpec((R, D), lambda i: (i, 0)),
        out_shape=jax.ShapeDtypeStruct((N, D), jnp.float32),
    )(agg, cnt32, z_prev)


def kernel(x, edge_index, Wl0, bl0, Wr0, Wl1, bl1, Wr1, Wl2, bl2, Wr2):
    src = edge_index[0]
    dst = edge_index[1]
    pad = EPAD - E
    src_p = jnp.concatenate([src, jnp.zeros((pad,), jnp.int32)])
    dst_p = jnp.concatenate([dst, jnp.full((pad,), N, jnp.int32)])
    src_s = src_p
    dst_s = dst_p.reshape(NS, CH, K)
    zrows = jnp.zeros((NPAD, H), jnp.float32)
    bl8s = [jnp.broadcast_to(b[None, :], (8, D)) for b in (bl0, bl1, bl2)]

    cnt32 = _sc_counts(dst_p).reshape(NC * NS, NPC)

    y2 = _tc_xw_halves(x, Wl0)
    agg = _sc_segsum(y2.reshape(2 * N, H), src_s, dst_s, zrows)
    z = _tc_xw_bias(x, Wr0, bl8s[0])
    y2 = _tc_y(agg, cnt32, z, Wl1)
    agg1 = _sc_segsum(y2.reshape(2 * N, H), src_s, dst_s, zrows)
    z = _tc_z(agg, cnt32, z, Wr1, bl8s[1])
    y2 = _tc_y(agg1, cnt32, z, Wl2)
    agg2 = _sc_segsum(y2.reshape(2 * N, H), src_s, dst_s, zrows)
    z = _tc_z(agg1, cnt32, z, Wr2, bl8s[2])
    return _tc_last(agg2, cnt32, z)


# speedup vs baseline: 1.0312x; 1.0312x over previous
import functools

import jax
import jax.numpy as jnp
from jax import lax
from jax.experimental import pallas as pl
from jax.experimental.pallas import tpu as pltpu
from jax.experimental.pallas import tpu_sc as plsc

N = 10000
E = 160000
D = 256
H = 128
NC = 2
NS = 16
L = 16

K = 128
CH = 80
EPT = CH * K
EPAD = NS * EPT
EPC = EPAD // (NC * NS)

NROW = 624
NPAD = 10112
NZ = NPAD // NS
NPC = 10240

R = 1024


def _sc_mesh():
    return plsc.VectorSubcoreMesh(core_axis_name="c", subcore_axis_name="s")


def _sc_counts(dst_p):
    @functools.partial(
        pl.kernel,
        out_type=jax.ShapeDtypeStruct((NC * NS * NPC,), jnp.float32),
        mesh=_sc_mesh(),
        scratch_types=[
            pltpu.VMEM((EPC,), jnp.int32),
            pltpu.VMEM((NPC,), jnp.float32),
        ],
        compiler_params=pltpu.CompilerParams(needs_layout_passes=False),
    )
    def k(dst_hbm, out_hbm, dstv, cntv):
        c = lax.axis_index("c")
        s = lax.axis_index("s")
        g = s * NC + c
        zeros = jnp.zeros((L,), jnp.float32)

        def zbody(i, carry):
            cntv[pl.ds(i * L, L)] = zeros
            return carry

        lax.fori_loop(0, NPC // L, zbody, 0)
        pltpu.sync_copy(dst_hbm.at[pl.ds(g * EPC, EPC)], dstv)
        ones = jnp.ones((L,), jnp.float32)

        def body(i, carry):
            idx = dstv[pl.ds(i * L, L)]
            plsc.addupdate_scatter(cntv, [idx], ones)
            return carry

        lax.fori_loop(0, EPC // L, body, 0)
        pltpu.sync_copy(cntv, out_hbm.at[pl.ds(g * NPC, NPC)])

    return k(dst_p)


def _sc_segsum(y2n, src_s, dst_s, zrows):
    @functools.partial(
        pl.kernel,
        out_type=jax.ShapeDtypeStruct((N, D), jnp.float32),
        mesh=_sc_mesh(),
        scratch_types=[
            pltpu.VMEM((EPT,), jnp.int32),
            pltpu.VMEM((CH, K), jnp.int32),
            pltpu.VMEM((K, H), jnp.float32),
            pltpu.VMEM_SHARED((NPAD, H), jnp.float32),
        ],
    )
    def k(y_hbm, src_hbm, dst_hbm, z_hbm, out_hbm, sall, dall, rows, acc):
        c = lax.axis_index("c")
        s = lax.axis_index("s")
        base = c * N

        pltpu.sync_copy(src_hbm.at[pl.ds(s * EPT, EPT)], sall)
        pltpu.sync_copy(dst_hbm.at[s], dall)
        pltpu.sync_copy(z_hbm.at[pl.ds(s * NZ, NZ)], acc.at[pl.ds(s * NZ, NZ)])

        def ab(i, carry):
            sl = pl.ds(i * L, L)
            sall[sl] = sall[sl] + base
            return carry

        lax.fori_loop(0, EPT // L, ab, 0)
        plsc.subcore_barrier()

        def step(j, carry):
            pltpu.sync_copy(y_hbm.at[sall.at[pl.ds(j * K, K)]], rows)
            pltpu.sync_copy(rows, acc.at[dall.at[j]], add=True)
            return carry

        lax.fori_loop(0, CH, step, 0)
        plsc.subcore_barrier()
        pltpu.sync_copy(
            acc.at[pl.ds(s * NROW, NROW)],
            out_hbm.at[pl.ds(s * NROW, NROW), pl.ds(c * H, H)],
        )

        @pl.when(s == 0)
        def _tail():
            pltpu.sync_copy(
                acc.at[pl.ds(NS * NROW, N - NS * NROW)],
                out_hbm.at[pl.ds(NS * NROW, N - NS * NROW), pl.ds(c * H, H)],
            )

    return k(y2n, src_s, dst_s, zrows)


def _tc_first(x, Wl, Wr, bl8):
    grid = (pl.cdiv(N, R),)

    def body(x_ref, wl_ref, wr_ref, b_ref, y_ref, z_ref):
        xb = x_ref[...]
        y = jnp.dot(xb, wl_ref[...], preferred_element_type=jnp.float32)
        y_ref[0] = y[:, :H]
        y_ref[1] = y[:, H:]
        z_ref[...] = (
            jnp.dot(xb, wr_ref[...], preferred_element_type=jnp.float32)
            + b_ref[0][None, :]
        )

    return pl.pallas_call(
        body,
        grid=grid,
        in_specs=[
            pl.BlockSpec((R, D), lambda i: (i, 0)),
            pl.BlockSpec((D, D), lambda i: (0, 0)),
            pl.BlockSpec((D, D), lambda i: (0, 0)),
            pl.BlockSpec((8, D), lambda i: (0, 0)),
        ],
        out_specs=[
            pl.BlockSpec((2, R, H), lambda i: (0, i, 0)),
            pl.BlockSpec((R, D), lambda i: (i, 0)),
        ],
        out_shape=[
            jax.ShapeDtypeStruct((2, N, H), jnp.float32),
            jax.ShapeDtypeStruct((N, D), jnp.float32),
        ],
    )(x, Wl, Wr, bl8)


def _tc_mid(agg, cnt32, z_prev, Wl, Wr, bl8):
    grid = (pl.cdiv(N, R),)

    def body(a_ref, c_ref, zp_ref, wl_ref, wr_ref, b_ref, y_ref, z_ref):
        cnt = jnp.sum(c_ref[...], axis=0)
        inv = 1.0 / jnp.maximum(cnt, 1.0)
        h = jnp.maximum(a_ref[...] * inv[:, None] + zp_ref[...], 0.0)
        y = jnp.dot(h, wl_ref[...], preferred_element_type=jnp.float32)
        y_ref[0] = y[:, :H]
        y_ref[1] = y[:, H:]
        z_ref[...] = (
            jnp.dot(h, wr_ref[...], preferred_element_type=jnp.float32)
            + b_ref[0][None, :]
        )

    return pl.pallas_call(
        body,
        grid=grid,
        in_specs=[
            pl.BlockSpec((R, D), lambda i: (i, 0)),
            pl.BlockSpec((NC * NS, R), lambda i: (0, i)),
            pl.BlockSpec((R, D), lambda i: (i, 0)),
            pl.BlockSpec((D, D), lambda i: (0, 0)),
            pl.BlockSpec((D, D), lambda i: (0, 0)),
            pl.BlockSpec((8, D), lambda i: (0, 0)),
        ],
        out_specs=[
            pl.BlockSpec((2, R, H), lambda i: (0, i, 0)),
            pl.BlockSpec((R, D), lambda i: (i, 0)),
        ],
        out_shape=[
            jax.ShapeDtypeStruct((2, N, H), jnp.float32),
            jax.ShapeDtypeStruct((N, D), jnp.float32),
        ],
    )(agg, cnt32, z_prev, Wl, Wr, bl8)


def _tc_last(agg, cnt32, z_prev):
    grid = (pl.cdiv(N, R),)

    def body(a_ref, c_ref, zp_ref, o_ref):
        cnt = jnp.sum(c_ref[...], axis=0)
        inv = 1.0 / jnp.maximum(cnt, 1.0)
        o_ref[...] = a_ref[...] * inv[:, None] + zp_ref[...]

    return pl.pallas_call(
        body,
        grid=grid,
        in_specs=[
            pl.BlockSpec((R, D), lambda i: (i, 0)),
            pl.BlockSpec((NC * NS, R), lambda i: (0, i)),
            pl.BlockSpec((R, D), lambda i: (i, 0)),
        ],
        out_specs=pl.BlockSpec((R, D), lambda i: (i, 0)),
        out_shape=jax.ShapeDtypeStruct((N, D), jnp.float32),
    )(agg, cnt32, z_prev)


def kernel(x, edge_index, Wl0, bl0, Wr0, Wl1, bl1, Wr1, Wl2, bl2, Wr2):
    src = edge_index[0]
    dst = edge_index[1]
    pad = EPAD - E
    src_p = jnp.concatenate([src, jnp.zeros((pad,), jnp.int32)])
    dst_p = jnp.concatenate([dst, jnp.full((pad,), N, jnp.int32)])
    src_s = src_p
    dst_s = dst_p.reshape(NS, CH, K)
    zrows = jnp.zeros((NPAD, H), jnp.float32)
    bl8s = [jnp.broadcast_to(b[None, :], (8, D)) for b in (bl0, bl1, bl2)]

    cnt32 = _sc_counts(dst_p).reshape(NC * NS, NPC)

    y2, z = _tc_first(x, Wl0, Wr0, bl8s[0])
    agg = _sc_segsum(y2.reshape(2 * N, H), src_s, dst_s, zrows)
    y2, z = _tc_mid(agg, cnt32, z, Wl1, Wr1, bl8s[1])
    agg = _sc_segsum(y2.reshape(2 * N, H), src_s, dst_s, zrows)
    y2, z = _tc_mid(agg, cnt32, z, Wl2, Wr2, bl8s[2])
    agg = _sc_segsum(y2.reshape(2 * N, H), src_s, dst_s, zrows)
    return _tc_last(agg, cnt32, z)


# speedup vs baseline: 1.3935x; 1.3513x over previous
import functools

import jax
import jax.numpy as jnp
from jax import lax
from jax.experimental import pallas as pl
from jax.experimental.pallas import tpu as pltpu
from jax.experimental.pallas import tpu_sc as plsc

N = 10000
E = 160000
D = 256
H = 128
NC = 2
NS = 16
L = 16

K = 128
CH = 79
EPT = CH * K
EPAD = NS * EPT
EPC = EPAD // (NC * NS)

NROW = 624
NPAD = 10240
NZ = NPAD // NS
NPC = 10240

R = 1024


def _sc_mesh():
    return plsc.VectorSubcoreMesh(core_axis_name="c", subcore_axis_name="s")


def _sc_counts(dst_p):
    @functools.partial(
        pl.kernel,
        out_type=jax.ShapeDtypeStruct((NC * NS * NPC,), jnp.float32),
        mesh=_sc_mesh(),
        scratch_types=[
            pltpu.VMEM((EPC,), jnp.int32),
            pltpu.VMEM((NPC,), jnp.float32),
        ],
        compiler_params=pltpu.CompilerParams(needs_layout_passes=False),
    )
    def k(dst_hbm, out_hbm, dstv, cntv):
        c = lax.axis_index("c")
        s = lax.axis_index("s")
        g = s * NC + c
        zeros = jnp.zeros((L,), jnp.float32)

        def zbody(i, carry):
            cntv[pl.ds(i * L, L)] = zeros
            return carry

        lax.fori_loop(0, NPC // L, zbody, 0)
        pltpu.sync_copy(dst_hbm.at[pl.ds(g * EPC, EPC)], dstv)
        ones = jnp.ones((L,), jnp.float32)

        def body(i, carry):
            idx = dstv[pl.ds(i * L, L)]
            plsc.addupdate_scatter(cntv, [idx], ones)
            return carry

        lax.fori_loop(0, EPC // L, body, 0)
        pltpu.sync_copy(cntv, out_hbm.at[pl.ds(g * NPC, NPC)])

    return k(dst_p)


def _sc_segsum(y2n, src_s, dst_s, zrows):
    @functools.partial(
        pl.kernel,
        out_type=jax.ShapeDtypeStruct((N, D), jnp.float32),
        mesh=_sc_mesh(),
        scratch_types=[
            pltpu.VMEM((EPT,), jnp.int32),
            pltpu.VMEM((CH, K), jnp.int32),
            pltpu.VMEM((K, H), jnp.float32),
            pltpu.VMEM_SHARED((NPAD, H), jnp.float32),
        ],
    )
    def k(y_hbm, src_hbm, dst_hbm, z_hbm, out_hbm, sall, dall, rows, acc):
        c = lax.axis_index("c")
        s = lax.axis_index("s")
        base = c * N

        pltpu.sync_copy(src_hbm.at[pl.ds(s * EPT, EPT)], sall)
        pltpu.sync_copy(dst_hbm.at[s], dall)
        pltpu.sync_copy(z_hbm.at[pl.ds(s * NZ, NZ)], acc.at[pl.ds(s * NZ, NZ)])

        def ab(i, carry):
            sl = pl.ds(i * L, L)
            sall[sl] = sall[sl] + base
            return carry

        lax.fori_loop(0, EPT // L, ab, 0)
        plsc.subcore_barrier()

        def step(j, carry):
            pltpu.sync_copy(y_hbm.at[sall.at[pl.ds(j * K, K)]], rows)
            pltpu.sync_copy(rows, acc.at[dall.at[j]], add=True)
            return carry

        lax.fori_loop(0, CH, step, 0)
        plsc.subcore_barrier()
        pltpu.sync_copy(
            acc.at[pl.ds(s * NROW, NROW)],
            out_hbm.at[pl.ds(s * NROW, NROW), pl.ds(c * H, H)],
        )

        @pl.when(s == 0)
        def _tail():
            pltpu.sync_copy(
                acc.at[pl.ds(NS * NROW, N - NS * NROW)],
                out_hbm.at[pl.ds(NS * NROW, N - NS * NROW), pl.ds(c * H, H)],
            )

    return k(y2n, src_s, dst_s, zrows)


def _tc_first(x, Wl, Wr, bl8):
    grid = (pl.cdiv(N, R),)

    def body(x_ref, wl_ref, wr_ref, b_ref, y_ref, z_ref):
        xb = x_ref[...]
        y = jnp.dot(xb, wl_ref[...], preferred_element_type=jnp.float32)
        y_ref[0] = y[:, :H]
        y_ref[1] = y[:, H:]
        z_ref[...] = (
            jnp.dot(xb, wr_ref[...], preferred_element_type=jnp.float32)
            + b_ref[0][None, :]
        )

    return pl.pallas_call(
        body,
        grid=grid,
        in_specs=[
            pl.BlockSpec((R, D), lambda i: (i, 0)),
            pl.BlockSpec((D, D), lambda i: (0, 0)),
            pl.BlockSpec((D, D), lambda i: (0, 0)),
            pl.BlockSpec((8, D), lambda i: (0, 0)),
        ],
        out_specs=[
            pl.BlockSpec((2, R, H), lambda i: (0, i, 0)),
            pl.BlockSpec((R, D), lambda i: (i, 0)),
        ],
        out_shape=[
            jax.ShapeDtypeStruct((2, N, H), jnp.float32),
            jax.ShapeDtypeStruct((N, D), jnp.float32),
        ],
    )(x, Wl, Wr, bl8)


def _tc_mid(agg, cnt32, z_prev, Wl, Wr, bl8):
    grid = (pl.cdiv(N, R),)

    def body(a_ref, c_ref, zp_ref, wl_ref, wr_ref, b_ref, y_ref, z_ref):
        cnt = jnp.sum(c_ref[...], axis=0)
        inv = 1.0 / jnp.maximum(cnt, 1.0)
        h = jnp.maximum(a_ref[...] * inv[:, None] + zp_ref[...], 0.0)
        y = jnp.dot(h, wl_ref[...], preferred_element_type=jnp.float32)
        y_ref[0] = y[:, :H]
        y_ref[1] = y[:, H:]
        z_ref[...] = (
            jnp.dot(h, wr_ref[...], preferred_element_type=jnp.float32)
            + b_ref[0][None, :]
        )

    return pl.pallas_call(
        body,
        grid=grid,
        in_specs=[
            pl.BlockSpec((R, D), lambda i: (i, 0)),
            pl.BlockSpec((NC * NS, R), lambda i: (0, i)),
            pl.BlockSpec((R, D), lambda i: (i, 0)),
            pl.BlockSpec((D, D), lambda i: (0, 0)),
            pl.BlockSpec((D, D), lambda i: (0, 0)),
            pl.BlockSpec((8, D), lambda i: (0, 0)),
        ],
        out_specs=[
            pl.BlockSpec((2, R, H), lambda i: (0, i, 0)),
            pl.BlockSpec((R, D), lambda i: (i, 0)),
        ],
        out_shape=[
            jax.ShapeDtypeStruct((2, N, H), jnp.float32),
            jax.ShapeDtypeStruct((N, D), jnp.float32),
        ],
    )(agg, cnt32, z_prev, Wl, Wr, bl8)


def _tc_last(agg, cnt32, z_prev):
    grid = (pl.cdiv(N, R),)

    def body(a_ref, c_ref, zp_ref, o_ref):
        cnt = jnp.sum(c_ref[...], axis=0)
        inv = 1.0 / jnp.maximum(cnt, 1.0)
        o_ref[...] = a_ref[...] * inv[:, None] + zp_ref[...]

    return pl.pallas_call(
        body,
        grid=grid,
        in_specs=[
            pl.BlockSpec((R, D), lambda i: (i, 0)),
            pl.BlockSpec((NC * NS, R), lambda i: (0, i)),
            pl.BlockSpec((R, D), lambda i: (i, 0)),
        ],
        out_specs=pl.BlockSpec((R, D), lambda i: (i, 0)),
        out_shape=jax.ShapeDtypeStruct((N, D), jnp.float32),
    )(agg, cnt32, z_prev)


def kernel(x, edge_index, Wl0, bl0, Wr0, Wl1, bl1, Wr1, Wl2, bl2, Wr2):
    src = edge_index[0]
    dst = edge_index[1]
    pad = EPAD - E
    src_p = jnp.concatenate([src, jnp.zeros((pad,), jnp.int32)])
    dst_p = jnp.concatenate([dst, jnp.full((pad,), N, jnp.int32)])
    src_s = src_p
    dst_s = dst_p.reshape(NS, CH, K)
    zrows = jnp.zeros((NPAD, H), jnp.float32)
    bl8s = [jnp.broadcast_to(b[None, :], (8, D)) for b in (bl0, bl1, bl2)]

    cnt32 = _sc_counts(dst_p).reshape(NC * NS, NPC)

    y2, z = _tc_first(x, Wl0, Wr0, bl8s[0])
    agg = _sc_segsum(y2.reshape(2 * N, H), src_s, dst_s, zrows)
    y2, z = _tc_mid(agg, cnt32, z, Wl1, Wr1, bl8s[1])
    agg = _sc_segsum(y2.reshape(2 * N, H), src_s, dst_s, zrows)
    y2, z = _tc_mid(agg, cnt32, z, Wl2, Wr2, bl8s[2])
    agg = _sc_segsum(y2.reshape(2 * N, H), src_s, dst_s, zrows)
    return _tc_last(agg, cnt32, z)


# speedup vs baseline: 1.9437x; 1.3949x over previous
import functools

import jax
import jax.numpy as jnp
from jax import lax
from jax.experimental import pallas as pl
from jax.experimental.pallas import tpu as pltpu
from jax.experimental.pallas import tpu_sc as plsc

N = 10000
E = 160000
D = 256
H = 128
NC = 2
NS = 16
L = 16

K = 128
CH = 79
EPT = CH * K
EPAD = NS * EPT
EPC = EPAD // (NC * NS)

NROW = 624
NPAD = 10240
NZ = NPAD // NS
NPC = 10240

R = 1024


def _sc_mesh():
    return plsc.VectorSubcoreMesh(core_axis_name="c", subcore_axis_name="s")


def _sc_counts(dst_p):
    @functools.partial(
        pl.kernel,
        out_type=jax.ShapeDtypeStruct((NC * NS * NPC,), jnp.float32),
        mesh=_sc_mesh(),
        scratch_types=[
            pltpu.VMEM((EPC,), jnp.int32),
            pltpu.VMEM((NPC,), jnp.float32),
        ],
        compiler_params=pltpu.CompilerParams(needs_layout_passes=False),
    )
    def k(dst_hbm, out_hbm, dstv, cntv):
        c = lax.axis_index("c")
        s = lax.axis_index("s")
        g = s * NC + c
        zeros = jnp.zeros((L,), jnp.float32)

        def zbody(i, carry):
            cntv[pl.ds(i * L, L)] = zeros
            return carry

        lax.fori_loop(0, NPC // L, zbody, 0)
        pltpu.sync_copy(dst_hbm.at[pl.ds(g * EPC, EPC)], dstv)
        ones = jnp.ones((L,), jnp.float32)

        def body(i, carry):
            idx = dstv[pl.ds(i * L, L)]
            plsc.addupdate_scatter(cntv, [idx], ones)
            return carry

        lax.fori_loop(0, EPC // L, body, 0)
        pltpu.sync_copy(cntv, out_hbm.at[pl.ds(g * NPC, NPC)])

    return k(dst_p)


def _sc_segsum(y2n, src_s, dst_s, zrows):
    @functools.partial(
        pl.kernel,
        out_type=jax.ShapeDtypeStruct((N, D), jnp.float32),
        mesh=_sc_mesh(),
        scratch_types=[
            pltpu.VMEM((EPT,), jnp.int32),
            pltpu.VMEM((CH, K), jnp.int32),
            pltpu.VMEM((K, H), jnp.float32),
            pltpu.VMEM_SHARED((NPAD, H), jnp.float32),
        ],
    )
    def k(y_hbm, src_hbm, dst_hbm, z_hbm, out_hbm, sall, dall, rows, acc):
        c = lax.axis_index("c")
        s = lax.axis_index("s")
        base = c * N

        pltpu.sync_copy(src_hbm.at[pl.ds(s * EPT, EPT)], sall)
        pltpu.sync_copy(dst_hbm.at[s], dall)
        pltpu.sync_copy(z_hbm.at[pl.ds(s * NZ, NZ)], acc.at[pl.ds(s * NZ, NZ)])

        def ab(i, carry):
            sl = pl.ds(i * L, L)
            sall[sl] = sall[sl] + base
            return carry

        lax.fori_loop(0, EPT // L, ab, 0)
        plsc.subcore_barrier()

        def step(j, carry):
            pltpu.sync_copy(y_hbm.at[sall.at[pl.ds(j * K, K)]], rows)
            pltpu.sync_copy(rows, acc.at[dall.at[j]], add=True)
            return carry

        lax.fori_loop(0, CH, step, 0)
        plsc.subcore_barrier()
        pltpu.sync_copy(
            acc.at[pl.ds(s * NROW, NROW)],
            out_hbm.at[pl.ds(s * NROW, NROW), pl.ds(c * H, H)],
        )

        @pl.when(s == 0)
        def _tail():
            pltpu.sync_copy(
                acc.at[pl.ds(NS * NROW, N - NS * NROW)],
                out_hbm.at[pl.ds(NS * NROW, N - NS * NROW), pl.ds(c * H, H)],
            )

    return k(y2n, src_s, dst_s, zrows)


def _tc_first(x, Wl, Wr, bl8):
    grid = (pl.cdiv(N, R),)

    def body(x_ref, wl_ref, wr_ref, b_ref, y_ref, z_ref):
        xb = x_ref[...]
        y = jnp.dot(xb, wl_ref[...], preferred_element_type=jnp.float32)
        y_ref[0] = y[:, :H]
        y_ref[1] = y[:, H:]
        z_ref[...] = (
            jnp.dot(xb, wr_ref[...], preferred_element_type=jnp.float32)
            + b_ref[0][None, :]
        )

    return pl.pallas_call(
        body,
        grid=grid,
        in_specs=[
            pl.BlockSpec((R, D), lambda i: (i, 0)),
            pl.BlockSpec((D, D), lambda i: (0, 0)),
            pl.BlockSpec((D, D), lambda i: (0, 0)),
            pl.BlockSpec((8, D), lambda i: (0, 0)),
        ],
        out_specs=[
            pl.BlockSpec((2, R, H), lambda i: (0, i, 0)),
            pl.BlockSpec((R, D), lambda i: (i, 0)),
        ],
        out_shape=[
            jax.ShapeDtypeStruct((2, N, H), jnp.float32),
            jax.ShapeDtypeStruct((N, D), jnp.float32),
        ],
    )(x, Wl, Wr, bl8)


def _tc_mid(agg, cnt32, z_prev, Wl, Wr, bl8):
    grid = (pl.cdiv(N, R),)

    def body(a_ref, c_ref, zp_ref, wl_ref, wr_ref, b_ref, y_ref, z_ref):
        cnt = jnp.sum(c_ref[...], axis=0)
        inv = 1.0 / jnp.maximum(cnt, 1.0)
        h = jnp.maximum(a_ref[...] * inv[:, None] + zp_ref[...], 0.0)
        y = jnp.dot(h, wl_ref[...], preferred_element_type=jnp.float32)
        y_ref[0] = y[:, :H]
        y_ref[1] = y[:, H:]
        z_ref[...] = (
            jnp.dot(h, wr_ref[...], preferred_element_type=jnp.float32)
            + b_ref[0][None, :]
        )

    return pl.pallas_call(
        body,
        grid=grid,
        in_specs=[
            pl.BlockSpec((R, D), lambda i: (i, 0)),
            pl.BlockSpec((NC * NS, R), lambda i: (0, i)),
            pl.BlockSpec((R, D), lambda i: (i, 0)),
            pl.BlockSpec((D, D), lambda i: (0, 0)),
            pl.BlockSpec((D, D), lambda i: (0, 0)),
            pl.BlockSpec((8, D), lambda i: (0, 0)),
        ],
        out_specs=[
            pl.BlockSpec((2, R, H), lambda i: (0, i, 0)),
            pl.BlockSpec((R, D), lambda i: (i, 0)),
        ],
        out_shape=[
            jax.ShapeDtypeStruct((2, N, H), jnp.float32),
            jax.ShapeDtypeStruct((N, D), jnp.float32),
        ],
    )(agg, cnt32, z_prev, Wl, Wr, bl8)


def _tc_last(agg, cnt32, z_prev):
    grid = (pl.cdiv(N, R),)

    def body(a_ref, c_ref, zp_ref, o_ref):
        cnt = jnp.sum(c_ref[...], axis=0)
        inv = 1.0 / jnp.maximum(cnt, 1.0)
        o_ref[...] = a_ref[...] * inv[:, None] + zp_ref[...]

    return pl.pallas_call(
        body,
        grid=grid,
        in_specs=[
            pl.BlockSpec((R, D), lambda i: (i, 0)),
            pl.BlockSpec((NC * NS, R), lambda i: (0, i)),
            pl.BlockSpec((R, D), lambda i: (i, 0)),
        ],
        out_specs=pl.BlockSpec((R, D), lambda i: (i, 0)),
        out_shape=jax.ShapeDtypeStruct((N, D), jnp.float32),
    )(agg, cnt32, z_prev)


def kernel(x, edge_index, Wl0, bl0, Wr0, Wl1, bl1, Wr1, Wl2, bl2, Wr2):
    src = edge_index[0]
    dst = edge_index[1]
    pad = EPAD - E
    pi = jnp.arange(pad, dtype=jnp.int32)
    src_p = jnp.concatenate([src, (pi * 97) % N])
    dst_p = jnp.concatenate([dst, N + (pi % (NPAD - N))])
    src_s = src_p
    dst_s = dst_p.reshape(NS, CH, K)
    zrows = jnp.zeros((NPAD, H), jnp.float32)
    bl8s = [jnp.broadcast_to(b[None, :], (8, D)) for b in (bl0, bl1, bl2)]

    cnt32 = _sc_counts(dst_p).reshape(NC * NS, NPC)

    y2, z = _tc_first(x, Wl0, Wr0, bl8s[0])
    agg = _sc_segsum(y2.reshape(2 * N, H), src_s, dst_s, zrows)
    y2, z = _tc_mid(agg, cnt32, z, Wl1, Wr1, bl8s[1])
    agg = _sc_segsum(y2.reshape(2 * N, H), src_s, dst_s, zrows)
    y2, z = _tc_mid(agg, cnt32, z, Wl2, Wr2, bl8s[2])
    agg = _sc_segsum(y2.reshape(2 * N, H), src_s, dst_s, zrows)
    return _tc_last(agg, cnt32, z)


# speedup vs baseline: 2.9380x; 1.5116x over previous
import functools

import jax
import jax.numpy as jnp
from jax import lax
from jax.experimental import pallas as pl
from jax.experimental.pallas import tpu as pltpu
from jax.experimental.pallas import tpu_sc as plsc

N = 10000
E = 160000
D = 256
H = 128
NC = 2
NS = 16
L = 16

K = 128
CH = 80
EPT = CH * K
EPAD = NS * EPT
EPC = EPAD // (NC * NS)

NROW = 624
NPAD = 10112
NZ = NPAD // NS
NPC = 10240

R = 1024


def _sc_mesh():
    return plsc.VectorSubcoreMesh(core_axis_name="c", subcore_axis_name="s")


def _sc_counts(dst_p):
    @functools.partial(
        pl.kernel,
        out_type=jax.ShapeDtypeStruct((NC * NS * NPC,), jnp.float32),
        mesh=_sc_mesh(),
        scratch_types=[
            pltpu.VMEM((EPC,), jnp.int32),
            pltpu.VMEM((NPC,), jnp.float32),
        ],
        compiler_params=pltpu.CompilerParams(needs_layout_passes=False),
    )
    def k(dst_hbm, out_hbm, dstv, cntv):
        c = lax.axis_index("c")
        s = lax.axis_index("s")
        g = s * NC + c
        zeros = jnp.zeros((L,), jnp.float32)

        def zbody(i, carry):
            cntv[pl.ds(i * L, L)] = zeros
            return carry

        lax.fori_loop(0, NPC // L, zbody, 0)
        pltpu.sync_copy(dst_hbm.at[pl.ds(g * EPC, EPC)], dstv)
        ones = jnp.ones((L,), jnp.float32)

        def body(i, carry):
            idx = dstv[pl.ds(i * L, L)]
            plsc.addupdate_scatter(cntv, [idx], ones)
            return carry

        lax.fori_loop(0, EPC // L, body, 0)
        pltpu.sync_copy(cntv, out_hbm.at[pl.ds(g * NPC, NPC)])

    return k(dst_p)


def _sc_segsum(y2n, src_s, dst_s, zrows):
    @functools.partial(
        pl.kernel,
        out_type=jax.ShapeDtypeStruct((N, D), jnp.float32),
        mesh=_sc_mesh(),
        scratch_types=[
            pltpu.VMEM((CH, K), jnp.int32),
            pltpu.VMEM((2, K), jnp.int32),
            pltpu.VMEM((K, H), jnp.float32),
            pltpu.VMEM((K, H), jnp.float32),
            pltpu.VMEM_SHARED((NPAD, H), jnp.float32),
            pltpu.SemaphoreType.DMA,
            pltpu.SemaphoreType.DMA,
            pltpu.SemaphoreType.DMA,
            pltpu.SemaphoreType.DMA,
        ],
    )
    def k(y_hbm, src_hbm, dst_hbm, z_hbm, out_hbm, dall, sx,
          rows0, rows1, acc, sg0, sg1, sx0, sx1):
        c = lax.axis_index("c")
        s = lax.axis_index("s")
        base = c * N
        rows = (rows0, rows1)
        sg = (sg0, sg1)
        sxs = (sx0, sx1)

        pltpu.sync_copy(dst_hbm.at[s], dall)
        pltpu.sync_copy(z_hbm.at[pl.ds(s * NZ, NZ)], acc.at[pl.ds(s * NZ, NZ)])
        plsc.subcore_barrier()

        def add_base(b):
            for q in range(K // L):
                sl = pl.ds(q * L, L)
                sx[b, sl] = sx[b, sl] + base

        for b in range(2):
            pltpu.sync_copy(src_hbm.at[s, b], sx.at[b])
            add_base(b)
            pltpu.async_copy(y_hbm.at[sx.at[b]], rows[b], sg[b])

        def step(jj, carry):
            for b in range(2):
                ch = 2 * jj + b
                pltpu.make_async_copy(y_hbm.at[sx.at[b]], rows[b], sg[b]).wait()
                pltpu.async_copy(src_hbm.at[s, ch + 2], sx.at[b], sxs[b])
                pltpu.sync_copy(rows[b], acc.at[dall.at[ch]], add=True)
                pltpu.make_async_copy(src_hbm.at[s, ch + 2], sx.at[b],
                                      sxs[b]).wait()
                add_base(b)
                pltpu.async_copy(y_hbm.at[sx.at[b]], rows[b], sg[b])
            return carry

        lax.fori_loop(0, CH // 2 - 1, step, 0)
        for b in range(2):
            ch = CH - 2 + b
            pltpu.make_async_copy(y_hbm.at[sx.at[b]], rows[b], sg[b]).wait()
            pltpu.sync_copy(rows[b], acc.at[dall.at[ch]], add=True)
        plsc.subcore_barrier()
        pltpu.sync_copy(
            acc.at[pl.ds(s * NROW, NROW)],
            out_hbm.at[pl.ds(s * NROW, NROW), pl.ds(c * H, H)],
        )

        @pl.when(s == 0)
        def _tail():
            pltpu.sync_copy(
                acc.at[pl.ds(NS * NROW, N - NS * NROW)],
                out_hbm.at[pl.ds(NS * NROW, N - NS * NROW), pl.ds(c * H, H)],
            )

    return k(y2n, src_s, dst_s, zrows)


def _tc_first(x, Wl, Wr, bl8):
    grid = (pl.cdiv(N, R),)

    def body(x_ref, wl_ref, wr_ref, b_ref, y_ref, z_ref):
        xb = x_ref[...]
        y = jnp.dot(xb, wl_ref[...], preferred_element_type=jnp.float32)
        y_ref[0] = y[:, :H]
        y_ref[1] = y[:, H:]
        z_ref[...] = (
            jnp.dot(xb, wr_ref[...], preferred_element_type=jnp.float32)
            + b_ref[0][None, :]
        )

    return pl.pallas_call(
        body,
        grid=grid,
        in_specs=[
            pl.BlockSpec((R, D), lambda i: (i, 0)),
            pl.BlockSpec((D, D), lambda i: (0, 0)),
            pl.BlockSpec((D, D), lambda i: (0, 0)),
            pl.BlockSpec((8, D), lambda i: (0, 0)),
        ],
        out_specs=[
            pl.BlockSpec((2, R, H), lambda i: (0, i, 0)),
            pl.BlockSpec((R, D), lambda i: (i, 0)),
        ],
        out_shape=[
            jax.ShapeDtypeStruct((2, N, H), jnp.float32),
            jax.ShapeDtypeStruct((N, D), jnp.float32),
        ],
    )(x, Wl, Wr, bl8)


def _tc_mid(agg, cnt32, z_prev, Wl, Wr, bl8):
    grid = (pl.cdiv(N, R),)

    def body(a_ref, c_ref, zp_ref, wl_ref, wr_ref, b_ref, y_ref, z_ref):
        cnt = jnp.sum(c_ref[...], axis=0)
        inv = 1.0 / jnp.maximum(cnt, 1.0)
        h = jnp.maximum(a_ref[...] * inv[:, None] + zp_ref[...], 0.0)
        y = jnp.dot(h, wl_ref[...], preferred_element_type=jnp.float32)
        y_ref[0] = y[:, :H]
        y_ref[1] = y[:, H:]
        z_ref[...] = (
            jnp.dot(h, wr_ref[...], preferred_element_type=jnp.float32)
            + b_ref[0][None, :]
        )

    return pl.pallas_call(
        body,
        grid=grid,
        in_specs=[
            pl.BlockSpec((R, D), lambda i: (i, 0)),
            pl.BlockSpec((NC * NS, R), lambda i: (0, i)),
            pl.BlockSpec((R, D), lambda i: (i, 0)),
            pl.BlockSpec((D, D), lambda i: (0, 0)),
            pl.BlockSpec((D, D), lambda i: (0, 0)),
            pl.BlockSpec((8, D), lambda i: (0, 0)),
        ],
        out_specs=[
            pl.BlockSpec((2, R, H), lambda i: (0, i, 0)),
            pl.BlockSpec((R, D), lambda i: (i, 0)),
        ],
        out_shape=[
            jax.ShapeDtypeStruct((2, N, H), jnp.float32),
            jax.ShapeDtypeStruct((N, D), jnp.float32),
        ],
    )(agg, cnt32, z_prev, Wl, Wr, bl8)


def _tc_last(agg, cnt32, z_prev):
    grid = (pl.cdiv(N, R),)

    def body(a_ref, c_ref, zp_ref, o_ref):
        cnt = jnp.sum(c_ref[...], axis=0)
        inv = 1.0 / jnp.maximum(cnt, 1.0)
        o_ref[...] = a_ref[...] * inv[:, None] + zp_ref[...]

    return pl.pallas_call(
        body,
        grid=grid,
        in_specs=[
            pl.BlockSpec((R, D), lambda i: (i, 0)),
            pl.BlockSpec((NC * NS, R), lambda i: (0, i)),
            pl.BlockSpec((R, D), lambda i: (i, 0)),
        ],
        out_specs=pl.BlockSpec((R, D), lambda i: (i, 0)),
        out_shape=jax.ShapeDtypeStruct((N, D), jnp.float32),
    )(agg, cnt32, z_prev)


def kernel(x, edge_index, Wl0, bl0, Wr0, Wl1, bl1, Wr1, Wl2, bl2, Wr2):
    src = edge_index[0]
    dst = edge_index[1]
    pad = EPAD - E
    pi = jnp.arange(pad, dtype=jnp.int32)
    src_p = jnp.concatenate([src, (pi * 97) % N])
    dst_p = jnp.concatenate([dst, N + (pi % (NPAD - N))])
    src_s = src_p.reshape(NS, CH, K)
    dst_s = dst_p.reshape(NS, CH, K)
    zrows = jnp.zeros((NPAD, H), jnp.float32)
    bl8s = [jnp.broadcast_to(b[None, :], (8, D)) for b in (bl0, bl1, bl2)]

    cnt32 = _sc_counts(dst_p).reshape(NC * NS, NPC)

    y2, z = _tc_first(x, Wl0, Wr0, bl8s[0])
    agg = _sc_segsum(y2.reshape(2 * N, H), src_s, dst_s, zrows)
    y2, z = _tc_mid(agg, cnt32, z, Wl1, Wr1, bl8s[1])
    agg = _sc_segsum(y2.reshape(2 * N, H), src_s, dst_s, zrows)
    y2, z = _tc_mid(agg, cnt32, z, Wl2, Wr2, bl8s[2])
    agg = _sc_segsum(y2.reshape(2 * N, H), src_s, dst_s, zrows)
    return _tc_last(agg, cnt32, z)
